# baseline (speedup 1.0000x reference)
import jax
import jax.numpy as jnp
from jax import lax

from jax.experimental import pallas as pl
from jax.experimental.pallas import tpu as pltpu
from jax.experimental.pallas import tpu_sc as plsc

N_FIELDS = 13
VOCAB = 100000
DIM = 16
BATCH = 16384

NC, NS = 2, 16
NW = NC * NS
BPW = BATCH // NW
NT = 2 * N_FIELDS
NBUF = 4
PRE = 2


def _body(xt_hbm, ut_hbm, it_hbm, out_hbm, xv, r0, r1, r2, r3,
          sg0, sg1, sg2, sg3, sw0, sw1, sw2, sw3):
    wid = lax.axis_index("s") * NC + lax.axis_index("c")
    base = wid * BPW
    rows = [r0, r1, r2, r3]
    sg = [sg0, sg1, sg2, sg3]
    sw = [sw0, sw1, sw2, sw3]

    pltpu.sync_copy(xt_hbm.at[:, pl.ds(base, BPW)], xv)

    gs = [None] * NBUF
    ws = [None] * NBUF

    def issue_gather(t):
        b = t % NBUF
        if ws[b] is not None:
            ws[b].wait()
        tab = ut_hbm if t < N_FIELDS else it_hbm
        f = t if t < N_FIELDS else t - N_FIELDS
        gs[b] = pltpu.async_copy(tab.at[f].at[xv.at[t]], rows[b], sg[b])

    for t in range(PRE):
        issue_gather(t)
    for t in range(NT):
        b = t % NBUF
        if t + PRE < NT:
            issue_gather(t + PRE)
        gs[b].wait()
        ws[b] = pltpu.async_copy(
            rows[b], out_hbm.at[pl.ds(base, BPW), pl.ds(t * DIM, DIM)], sw[b])
    for b in range(NBUF):
        if ws[b] is not None:
            ws[b].wait()


def _as_row_major(t):
    zero = lax.optimization_barrier(jnp.int32(0))
    ti = lax.bitcast_convert_type(t, jnp.int32) ^ zero
    return lax.bitcast_convert_type(ti, jnp.float32)


@jax.jit
def kernel(x, user_tables, item_tables):
    xt = x.astype(jnp.int32).T
    user_tables = _as_row_major(user_tables)
    item_tables = _as_row_major(item_tables)
    mesh = plsc.VectorSubcoreMesh(
        core_axis_name="c", subcore_axis_name="s",
        num_cores=NC, num_subcores=NS)
    return pl.kernel(
        _body,
        out_type=jax.ShapeDtypeStruct((BATCH, NT * DIM), jnp.float32),
        mesh=mesh,
        compiler_params=pltpu.CompilerParams(use_tc_tiling_on_sc=False),
        scratch_types=[
            pltpu.VMEM((NT, BPW), jnp.int32),
            pltpu.VMEM((BPW, DIM), jnp.float32),
            pltpu.VMEM((BPW, DIM), jnp.float32),
            pltpu.VMEM((BPW, DIM), jnp.float32),
            pltpu.VMEM((BPW, DIM), jnp.float32),
            pltpu.SemaphoreType.DMA,
            pltpu.SemaphoreType.DMA,
            pltpu.SemaphoreType.DMA,
            pltpu.SemaphoreType.DMA,
            pltpu.SemaphoreType.DMA,
            pltpu.SemaphoreType.DMA,
            pltpu.SemaphoreType.DMA,
            pltpu.SemaphoreType.DMA,
        ],
    )(xt, user_tables, item_tables)


# speedup vs baseline: 1.0010x; 1.0010x over previous
import jax
import jax.numpy as jnp
from jax import lax

from jax.experimental import pallas as pl
from jax.experimental.pallas import tpu as pltpu
from jax.experimental.pallas import tpu_sc as plsc

N_FIELDS = 13
VOCAB = 100000
DIM = 16
BATCH = 16384

NC, NS = 2, 16
NW = NC * NS
BPW = BATCH // NW
NT = 2 * N_FIELDS
NBUF = 8
PRE = 6


def _body(xt_hbm, ut_hbm, it_hbm, out_hbm, xv, *scr):
    wid = lax.axis_index("s") * NC + lax.axis_index("c")
    base = wid * BPW
    rows = list(scr[:NBUF])
    sg = list(scr[NBUF:2 * NBUF])
    sw = list(scr[2 * NBUF:3 * NBUF])

    pltpu.sync_copy(xt_hbm.at[:, pl.ds(base, BPW)], xv)

    gs = [None] * NBUF
    ws = [None] * NBUF

    def issue_gather(t):
        b = t % NBUF
        if ws[b] is not None:
            ws[b].wait()
        tab = ut_hbm if t < N_FIELDS else it_hbm
        f = t if t < N_FIELDS else t - N_FIELDS
        gs[b] = pltpu.async_copy(tab.at[f].at[xv.at[t]], rows[b], sg[b])

    for t in range(PRE):
        issue_gather(t)
    for t in range(NT):
        b = t % NBUF
        if t + PRE < NT:
            issue_gather(t + PRE)
        gs[b].wait()
        ws[b] = pltpu.async_copy(
            rows[b], out_hbm.at[pl.ds(base, BPW), pl.ds(t * DIM, DIM)], sw[b])
    for b in range(NBUF):
        if ws[b] is not None:
            ws[b].wait()


def _as_row_major(t):
    zero = lax.optimization_barrier(jnp.int32(0))
    ti = lax.bitcast_convert_type(t, jnp.int32) ^ zero
    return lax.bitcast_convert_type(ti, jnp.float32)


@jax.jit
def kernel(x, user_tables, item_tables):
    xt = x.astype(jnp.int32).T
    user_tables = _as_row_major(user_tables)
    item_tables = _as_row_major(item_tables)
    mesh = plsc.VectorSubcoreMesh(
        core_axis_name="c", subcore_axis_name="s",
        num_cores=NC, num_subcores=NS)
    return pl.kernel(
        _body,
        out_type=jax.ShapeDtypeStruct((BATCH, NT * DIM), jnp.float32),
        mesh=mesh,
        compiler_params=pltpu.CompilerParams(use_tc_tiling_on_sc=False),
        scratch_types=(
            [pltpu.VMEM((NT, BPW), jnp.int32)]
            + [pltpu.VMEM((BPW, DIM), jnp.float32)] * NBUF
            + [pltpu.SemaphoreType.DMA] * NBUF
            + [pltpu.SemaphoreType.DMA] * NBUF
        ),
    )(xt, user_tables, item_tables)


# speedup vs baseline: 1.9129x; 1.9110x over previous
import jax
import jax.numpy as jnp
from jax import lax

from jax.experimental import pallas as pl
from jax.experimental.pallas import tpu as pltpu
from jax.experimental.pallas import tpu_sc as plsc

N_FIELDS = 13
VOCAB = 100000
DIM = 16
BATCH = 16384

NC, NS = 2, 16
NW = NC * NS
BPW = BATCH // NW
NT = 2 * N_FIELDS
NBUF = 8
PRE = 6


def _body(xt_hbm, ut_hbm, it_hbm, out_hbm, xv, *scr):
    wid = lax.axis_index("s") * NC + lax.axis_index("c")
    base = wid * BPW
    rows = list(scr[:NBUF])
    sg = list(scr[NBUF:2 * NBUF])
    sw = list(scr[2 * NBUF:3 * NBUF])

    pltpu.sync_copy(xt_hbm.at[:, pl.ds(base, BPW)], xv)

    gs = [None] * NBUF
    ws = [None] * NBUF

    def issue_gather(t):
        b = t % NBUF
        if ws[b] is not None:
            ws[b].wait()
        tab = ut_hbm if t < N_FIELDS else it_hbm
        f = t if t < N_FIELDS else t - N_FIELDS
        gs[b] = pltpu.async_copy(tab.at[f].at[xv.at[t]], rows[b], sg[b])

    for t in range(PRE):
        issue_gather(t)
    for t in range(NT):
        b = t % NBUF
        if t + PRE < NT:
            issue_gather(t + PRE)
        gs[b].wait()
        ws[b] = pltpu.async_copy(
            rows[b], out_hbm.at[pl.ds(base, BPW), pl.ds(t * DIM, DIM)], sw[b])
    for b in range(NBUF):
        if ws[b] is not None:
            ws[b].wait()


def _as_row_major(t):
    zero = lax.optimization_barrier(jnp.int32(0))
    ti = lax.bitcast_convert_type(t, jnp.int32) ^ zero
    return lax.bitcast_convert_type(ti, jnp.float32)


@jax.jit
def kernel(x, user_tables, item_tables):
    xt = x.astype(jnp.int32).T
    mesh = plsc.VectorSubcoreMesh(
        core_axis_name="c", subcore_axis_name="s",
        num_cores=NC, num_subcores=NS)
    return pl.kernel(
        _body,
        out_type=jax.ShapeDtypeStruct((BATCH, NT * DIM), jnp.float32),
        mesh=mesh,
        compiler_params=pltpu.CompilerParams(use_tc_tiling_on_sc=False),
        scratch_types=(
            [pltpu.VMEM((NT, BPW), jnp.int32)]
            + [pltpu.VMEM((BPW, DIM), jnp.float32)] * NBUF
            + [pltpu.SemaphoreType.DMA] * NBUF
            + [pltpu.SemaphoreType.DMA] * NBUF
        ),
    )(xt, user_tables, item_tables)
